# baseline (speedup 1.0000x reference)
import functools

import jax
import jax.numpy as jnp
from jax import lax
from jax.experimental import pallas as pl
from jax.experimental.pallas import tpu as pltpu
from jax.experimental.pallas import tpu_sc as plsc

_ALPHA = 0.2
_NEG = -9e15


def _make_sc_gather(n_rows, dim):
    info = plsc.get_sparse_core_info()
    nc, ns = info.num_cores, info.num_subcores
    nw = nc * ns
    assert n_rows % nw == 0
    b_per_w = n_rows // nw
    ch = 400
    assert b_per_w % ch == 0
    n_chunks = b_per_w // ch
    mesh = plsc.VectorSubcoreMesh(core_axis_name="c", subcore_axis_name="s")

    @functools.partial(
        pl.kernel,
        mesh=mesh,
        out_type=jax.ShapeDtypeStruct((n_rows, dim), jnp.float32),
        scratch_types=[
            pltpu.VMEM((b_per_w,), jnp.int32),
            pltpu.VMEM((2, ch, dim), jnp.float32),
            pltpu.SemaphoreType.DMA,
            pltpu.SemaphoreType.DMA,
        ],
    )
    def gather_kernel(table_hbm, idx_hbm, out_hbm, idx_v, rows_v, sem0, sem1):
        wid = lax.axis_index("s") * nc + lax.axis_index("c")
        base = wid * b_per_w
        sems = [sem0, sem1]
        pltpu.sync_copy(idx_hbm.at[pl.ds(base, b_per_w)], idx_v)
        copies = [None, None]
        copies[0] = pltpu.async_copy(
            table_hbm.at[idx_v.at[pl.ds(0, ch)]], rows_v.at[0], sems[0]
        )
        for c in range(n_chunks):
            nxt = c + 1
            if nxt < n_chunks:
                copies[nxt % 2] = pltpu.async_copy(
                    table_hbm.at[idx_v.at[pl.ds(nxt * ch, ch)]],
                    rows_v.at[nxt % 2],
                    sems[nxt % 2],
                )
            copies[c % 2].wait()
            pltpu.sync_copy(rows_v.at[c % 2], out_hbm.at[pl.ds(base + c * ch, ch)])

    return gather_kernel


def _attn_body(h_ref, adj_ref, a_ref, out_ref):
    h = h_ref[0]
    adj = adj_ref[0]
    a = a_ref[...]

    att = jnp.full(adj.shape, _NEG, dtype=jnp.float32)
    for k in range(4):
        ha = h * a[k, :][None, :]
        e = lax.dot_general(
            ha, h, (((1,), (1,)), ((), ())), preferred_element_type=jnp.float32
        )
        e = jnp.where(e >= 0, e, _ALPHA * e)
        att = jnp.where(adj == (k + 1), e, att)

    m = jnp.max(att, axis=-1, keepdims=True)
    p = jnp.exp(att - m)
    att = p / jnp.sum(p, axis=-1, keepdims=True)
    out_ref[0] = lax.dot_general(
        att, h, (((1,), (0,)), ((), ())), preferred_element_type=jnp.float32
    )


def _attention_tc(h, adj, a_stack):
    b, l, d = h.shape
    return pl.pallas_call(
        _attn_body,
        grid=(b,),
        in_specs=[
            pl.BlockSpec((1, l, d), lambda i: (i, 0, 0)),
            pl.BlockSpec((1, l, l), lambda i: (i, 0, 0)),
            pl.BlockSpec((8, d), lambda i: (0, 0)),
        ],
        out_specs=pl.BlockSpec((1, l, d), lambda i: (i, 0, 0)),
        out_shape=jax.ShapeDtypeStruct((b, l, d), jnp.float32),
    )(h, adj, a_stack)


def kernel(inputs, adj, mask_item, item, embedding, a_0, a_1, a_2, a_3):
    b, l = inputs.shape
    _, dim = embedding.shape
    idx_flat = inputs.reshape(-1).astype(jnp.int32)

    gather = _make_sc_gather(b * l, dim)
    h_flat = gather(embedding, idx_flat)
    h = h_flat.reshape(b, l, dim)

    a_stack = jnp.concatenate(
        [a_0.T, a_1.T, a_2.T, a_3.T, jnp.zeros((4, dim), jnp.float32)], axis=0
    )
    return _attention_tc(h, adj, a_stack)
